# baseline (speedup 1.0000x reference)
from functools import partial

import jax
import jax.numpy as jnp
from jax.experimental import pallas as pl
from jax.experimental.pallas import tpu as pltpu

_EPS = 1e-5
_VMEM_LIMIT = 32 * 1024 * 1024


def _chunk_body(x_ref, gamma_ref, beta_ref, w_ref, b_ref, o_ref, *,
                inv_n, gsize):
    x = x_ref[0]
    Cc = x.shape[0]

    s1 = jnp.sum(x, axis=-1, keepdims=True)
    s2 = jnp.sum(x * x, axis=-1, keepdims=True)

    row = jax.lax.broadcasted_iota(jnp.int32, (Cc, Cc), 0) // gsize
    col = jax.lax.broadcasted_iota(jnp.int32, (Cc, Cc), 1) // gsize
    mask = (row == col).astype(jnp.float32)
    s12 = jnp.concatenate([s1, s2], axis=1)
    gs = jnp.dot(mask, s12, preferred_element_type=jnp.float32,
                 precision=jax.lax.Precision.HIGHEST)

    mean = gs[:, 0:1] * inv_n
    ex2 = gs[:, 1:2] * inv_n
    rstd = jax.lax.rsqrt(ex2 - mean * mean + _EPS)
    scale = gamma_ref[...] * rstd
    shift = beta_ref[...] - mean * scale

    y = (x * scale + shift).astype(jnp.bfloat16)
    part = jnp.dot(w_ref[0], y, preferred_element_type=jnp.float32)

    cc = pl.program_id(1)

    @pl.when(cc == 0)
    def _init():
        o_ref[0] = (part + b_ref[...]).astype(o_ref.dtype)

    @pl.when(cc != 0)
    def _acc():
        o_ref[0] = o_ref[0] + part.astype(o_ref.dtype)


def kernel(x, gamma, beta, w, b):
    B, C, H, W = x.shape
    HW = H * W
    num_groups = C // 4 if C % 4 == 0 else C
    gsize = C // num_groups
    inv_n = 1.0 / float(gsize * HW)

    nc = 4 if (C % 4 == 0 and (C // 4) % gsize == 0) else 1
    Cc = C // nc

    xf = x.reshape(B, C, HW)
    gamma2 = jnp.asarray(gamma, jnp.float32).reshape(C, 1)
    beta2 = jnp.asarray(beta, jnp.float32).reshape(C, 1)
    b2 = jnp.asarray(b, jnp.float32).reshape(C, 1)
    wbf = jnp.asarray(w).astype(jnp.bfloat16).reshape(C, nc, Cc).transpose(1, 0, 2)

    out = pl.pallas_call(
        partial(_chunk_body, inv_n=inv_n, gsize=gsize),
        out_shape=jax.ShapeDtypeStruct((B, C, HW), x.dtype),
        grid=(B, nc),
        in_specs=[
            pl.BlockSpec((1, Cc, HW), lambda bb, cc: (bb, cc, 0)),
            pl.BlockSpec((Cc, 1), lambda bb, cc: (cc, 0)),
            pl.BlockSpec((Cc, 1), lambda bb, cc: (cc, 0)),
            pl.BlockSpec((1, C, Cc), lambda bb, cc: (cc, 0, 0)),
            pl.BlockSpec((C, 1), lambda bb, cc: (0, 0)),
        ],
        out_specs=pl.BlockSpec((1, C, HW), lambda bb, cc: (bb, 0, 0)),
        compiler_params=pltpu.CompilerParams(
            dimension_semantics=("parallel", "arbitrary"),
            vmem_limit_bytes=_VMEM_LIMIT),
    )(xf, gamma2, beta2, wbf, b2)

    return out.reshape(B, C, H, W)
